# Initial kernel scaffold; baseline (speedup 1.0000x reference)
import jax
import jax.numpy as jnp
from jax.experimental import pallas as pl


def kernel(x, edge_index, Wl0, Wr0, b0, Wl1, Wr1, b1, Wl2, Wr2, b2):
    raise NotImplementedError("write your pallas kernel here")


# speedup vs baseline: 3.1767x; 3.1767x over previous
import functools

import jax
import jax.numpy as jnp
from jax import lax
from jax.experimental import pallas as pl
from jax.experimental.pallas import tpu as pltpu
from jax.experimental.pallas import tpu_sc as plsc

N = 50000
E = 800000
H = 128
NC = 2
NS = 16
BATCH = 128
N_PAD = 50176
RPT = N_PAD // NS
ZR = 112


def _zero_acc(zbuf, acc, s, ncols):
    @pl.loop(0, ZR)
    def _(j):
        for k in range(ncols // 16):
            zbuf[j, pl.ds(k * 16, 16)] = jnp.zeros((16,), jnp.float32)

    @pl.loop(0, RPT // ZR)
    def _(j):
        pltpu.sync_copy(zbuf, acc.at[pl.ds(s * RPT + j * ZR, ZR)])


def _edge_pass(table, src_hbm, dst_hbm, acc, isrc, idst, rows, base, nb):
    @pl.loop(0, nb)
    def _(b):
        off = base + b * BATCH
        pltpu.sync_copy(src_hbm.at[pl.ds(off, BATCH)], isrc)
        pltpu.sync_copy(table.at[isrc], rows)
        pltpu.sync_copy(dst_hbm.at[pl.ds(off, BATCH)], idst)
        pltpu.sync_copy(rows, acc.at[idst], add=True)


def _sc_agg16(x16, src, dst):
    ept = E // (NC * NS)
    nb = ept // BATCH
    tail = ept - nb * BATCH
    mesh = plsc.VectorSubcoreMesh(core_axis_name="c", subcore_axis_name="s")

    @functools.partial(
        pl.kernel,
        out_type=jax.ShapeDtypeStruct((NC, N_PAD, 16), jnp.float32),
        mesh=mesh,
        compiler_params=pltpu.CompilerParams(use_tc_tiling_on_sc=False),
        scratch_types=[
            pltpu.VMEM((BATCH,), jnp.int32),
            pltpu.VMEM((BATCH,), jnp.int32),
            pltpu.VMEM((BATCH, 16), jnp.float32),
            pltpu.VMEM((tail,), jnp.int32),
            pltpu.VMEM((tail,), jnp.int32),
            pltpu.VMEM((tail, 16), jnp.float32),
            pltpu.VMEM((ZR, 16), jnp.float32),
            pltpu.VMEM_SHARED((N_PAD, 16), jnp.float32),
        ],
    )
    def k(x_hbm, src_hbm, dst_hbm, out_hbm,
          isrc, idst, rows, isrc_t, idst_t, rows_t, zbuf, acc):
        c = lax.axis_index("c")
        s = lax.axis_index("s")
        _zero_acc(zbuf, acc, s, 16)
        plsc.subcore_barrier()
        base = c * (E // NC) + s * ept
        _edge_pass(x_hbm, src_hbm, dst_hbm, acc, isrc, idst, rows, base, nb)
        off = base + nb * BATCH
        pltpu.sync_copy(src_hbm.at[pl.ds(off, tail)], isrc_t)
        pltpu.sync_copy(x_hbm.at[isrc_t], rows_t)
        pltpu.sync_copy(dst_hbm.at[pl.ds(off, tail)], idst_t)
        pltpu.sync_copy(rows_t, acc.at[idst_t], add=True)
        plsc.subcore_barrier()
        pltpu.sync_copy(acc.at[pl.ds(s * RPT, RPT)],
                        out_hbm.at[c].at[pl.ds(s * RPT, RPT)])

    return k(x16, src, dst)


def _sc_agg128(h4, src, dst):
    ept = E // NS
    nb = ept // BATCH
    tail = ept - nb * BATCH
    mesh = plsc.VectorSubcoreMesh(core_axis_name="c", subcore_axis_name="s")

    @functools.partial(
        pl.kernel,
        out_type=jax.ShapeDtypeStruct((4, N_PAD, 32), jnp.float32),
        mesh=mesh,
        compiler_params=pltpu.CompilerParams(use_tc_tiling_on_sc=False),
        scratch_types=[
            pltpu.VMEM((BATCH,), jnp.int32),
            pltpu.VMEM((BATCH,), jnp.int32),
            pltpu.VMEM((BATCH, 32), jnp.float32),
            pltpu.VMEM((tail,), jnp.int32),
            pltpu.VMEM((tail,), jnp.int32),
            pltpu.VMEM((tail, 32), jnp.float32),
            pltpu.VMEM((ZR, 32), jnp.float32),
            pltpu.VMEM_SHARED((N_PAD, 32), jnp.float32),
        ],
    )
    def k(h_hbm, src_hbm, dst_hbm, out_hbm,
          isrc, idst, rows, isrc_t, idst_t, rows_t, zbuf, acc):
        c = lax.axis_index("c")
        s = lax.axis_index("s")
        for cc_local in range(2):
            cc = c * 2 + cc_local
            _zero_acc(zbuf, acc, s, 32)
            plsc.subcore_barrier()
            base = s * ept
            table = h_hbm.at[cc]
            _edge_pass(table, src_hbm, dst_hbm, acc, isrc, idst, rows,
                       base, nb)
            off = base + nb * BATCH
            pltpu.sync_copy(src_hbm.at[pl.ds(off, tail)], isrc_t)
            pltpu.sync_copy(table.at[isrc_t], rows_t)
            pltpu.sync_copy(dst_hbm.at[pl.ds(off, tail)], idst_t)
            pltpu.sync_copy(rows_t, acc.at[idst_t], add=True)
            plsc.subcore_barrier()
            pltpu.sync_copy(acc.at[pl.ds(s * RPT, RPT)],
                            out_hbm.at[cc].at[pl.ds(s * RPT, RPT)])
            plsc.subcore_barrier()

    return k(h4, src, dst)


BN = 2000


def _tc_layer0(agg0, x16, wl_t, wr_t, b):
    def body(agg_ref, x_ref, wl_ref, wr_ref, b_ref, h4_ref):
        sarr = agg_ref[0] + agg_ref[1]
        cnt = sarr[:, 9:10]
        invc = 1.0 / jnp.maximum(cnt, 1.0)
        m = sarr * invc
        z = jnp.dot(m, wl_ref[...], preferred_element_type=jnp.float32)
        z = z + jnp.dot(x_ref[...], wr_ref[...],
                        preferred_element_type=jnp.float32)
        z = z + b_ref[...]
        h = jnp.maximum(z, 0.0)
        for c4 in range(4):
            h4_ref[c4] = h[:, c4 * 32:(c4 + 1) * 32]

    return pl.pallas_call(
        body,
        grid=(N // BN,),
        in_specs=[
            pl.BlockSpec((2, BN, 16), lambda i: (0, i, 0)),
            pl.BlockSpec((BN, 16), lambda i: (i, 0)),
            pl.BlockSpec((16, H), lambda i: (0, 0)),
            pl.BlockSpec((16, H), lambda i: (0, 0)),
            pl.BlockSpec((1, H), lambda i: (0, 0)),
        ],
        out_specs=pl.BlockSpec((4, BN, 32), lambda i: (0, i, 0)),
        out_shape=jax.ShapeDtypeStruct((4, N, 32), jnp.float32),
    )(agg0, x16, wl_t, wr_t, b)


def _tc_layer(agg4, h4, agg0, wl_t, wr_t, b, relu, out4):
    def body(agg_ref, h_ref, a0_ref, wl_ref, wr_ref, b_ref, out_ref):
        cnt = a0_ref[0, :, 9:10] + a0_ref[1, :, 9:10]
        invc = 1.0 / jnp.maximum(cnt, 1.0)
        m = jnp.concatenate([agg_ref[c] for c in range(4)], axis=1) * invc
        hp = jnp.concatenate([h_ref[c] for c in range(4)], axis=1)
        z = jnp.dot(m, wl_ref[...], preferred_element_type=jnp.float32)
        z = z + jnp.dot(hp, wr_ref[...], preferred_element_type=jnp.float32)
        z = z + b_ref[...]
        if relu:
            z = jnp.maximum(z, 0.0)
        if out4:
            for c4 in range(4):
                out_ref[c4] = z[:, c4 * 32:(c4 + 1) * 32]
        else:
            out_ref[...] = z

    if out4:
        out_spec = pl.BlockSpec((4, BN, 32), lambda i: (0, i, 0))
        out_shape = jax.ShapeDtypeStruct((4, N, 32), jnp.float32)
    else:
        out_spec = pl.BlockSpec((BN, H), lambda i: (i, 0))
        out_shape = jax.ShapeDtypeStruct((N, H), jnp.float32)

    return pl.pallas_call(
        body,
        grid=(N // BN,),
        in_specs=[
            pl.BlockSpec((4, BN, 32), lambda i: (0, i, 0)),
            pl.BlockSpec((4, BN, 32), lambda i: (0, i, 0)),
            pl.BlockSpec((2, BN, 16), lambda i: (0, i, 0)),
            pl.BlockSpec((H, H), lambda i: (0, 0)),
            pl.BlockSpec((H, H), lambda i: (0, 0)),
            pl.BlockSpec((1, H), lambda i: (0, 0)),
        ],
        out_specs=out_spec,
        out_shape=out_shape,
    )(agg4, h4, agg0, wl_t, wr_t, b)


def kernel(x, edge_index, Wl0, Wr0, b0, Wl1, Wr1, b1, Wl2, Wr2, b2):
    src = edge_index[0]
    dst = edge_index[1]
    x16 = jnp.concatenate(
        [x, jnp.ones((N, 1), jnp.float32), jnp.zeros((N, 6), jnp.float32)],
        axis=1)
    wl0_t = jnp.pad(Wl0, ((0, 0), (0, 7))).T
    wr0_t = jnp.pad(Wr0, ((0, 0), (0, 7))).T
    b0r = b0.reshape(1, H)

    agg0 = _sc_agg16(x16, src, dst)
    h1_4 = _tc_layer0(agg0, x16, wl0_t, wr0_t, b0r)
    agg1 = _sc_agg128(h1_4, src, dst)
    h2_4 = _tc_layer(agg1, h1_4, agg0, Wl1.T, Wr1.T, b1.reshape(1, H),
                     relu=True, out4=True)
    agg2 = _sc_agg128(h2_4, src, dst)
    out = _tc_layer(agg2, h2_4, agg0, Wl2.T, Wr2.T, b2.reshape(1, H),
                    relu=False, out4=False)
    return out


# speedup vs baseline: 10.0651x; 3.1684x over previous
import functools

import jax
import jax.numpy as jnp
from jax import lax
from jax.experimental import pallas as pl
from jax.experimental.pallas import tpu as pltpu
from jax.experimental.pallas import tpu_sc as plsc

N = 50000
E = 800000
H = 128
NC = 2
NS = 16
NBUF = 4
N_PAD = 50176
RPT = N_PAD // NS
ZR = 56

_SC_PARAMS = pltpu.CompilerParams(use_tc_tiling_on_sc=False)


def _zero_acc(zbuf, acc, s, ncols):
    @pl.loop(0, ZR)
    def _(j):
        for k in range(ncols // 16):
            zbuf[j, pl.ds(k * 16, 16)] = jnp.zeros((16,), jnp.float32)

    @pl.loop(0, RPT // ZR)
    def _(j):
        pltpu.sync_copy(zbuf, acc.at[pl.ds(s * RPT + j * ZR, ZR)])


def _edge_pass(table, ei, acc, slots, base, nb, B):
    def I1(b, k):
        esrc, edst, rows, s1, s2, sg, ss = slots[k]
        pltpu.async_copy(ei.at[0, pl.ds(base + b * B, B)], esrc, s1)

    def wI1(k):
        esrc, edst, rows, s1, s2, sg, ss = slots[k]
        pltpu.make_async_copy(ei.at[0, pl.ds(base, B)], esrc, s1).wait()

    def I2(b, k):
        esrc, edst, rows, s1, s2, sg, ss = slots[k]
        pltpu.async_copy(ei.at[1, pl.ds(base + b * B, B)], edst, s2)

    def wI2(k):
        esrc, edst, rows, s1, s2, sg, ss = slots[k]
        pltpu.make_async_copy(ei.at[1, pl.ds(base, B)], edst, s2).wait()

    def G(k):
        esrc, edst, rows, s1, s2, sg, ss = slots[k]
        pltpu.async_copy(table.at[esrc], rows, sg)

    def wG(k):
        esrc, edst, rows, s1, s2, sg, ss = slots[k]
        pltpu.make_async_copy(table.at[esrc], rows, sg).wait()

    def S(k):
        esrc, edst, rows, s1, s2, sg, ss = slots[k]
        pltpu.async_copy(rows, acc.at[edst], ss, add=True)

    def wS(k):
        esrc, edst, rows, s1, s2, sg, ss = slots[k]
        pltpu.make_async_copy(rows, acc.at[edst], ss).wait()

    def step(tv, k0, g_wsp, g_next, g_i3):
        k1 = (k0 + 1) % NBUF
        k3 = (k0 + 3) % NBUF
        if g_next:
            wI1(k1)
            G(k1)
            I2(tv + 1, k1)
        wG(k0)
        wI2(k0)
        if g_wsp:
            wS(k3)
        S(k0)
        if g_i3:
            I1(tv + 3, k3)

    for b in range(min(3, nb)):
        I1(b, b % NBUF)
    I2(0, 0)
    wI1(0)
    G(0)

    front = min(3, nb)
    main = max(0, nb - 3 - front) // NBUF * NBUF
    for t in range(front):
        step(t, t % NBUF, t >= 1, t + 1 < nb, t + 3 < nb)
    if main > 0:
        @pl.loop(0, main // NBUF)
        def _(g):
            tv = front + g * NBUF
            for j in range(NBUF):
                step(tv + j, (front + j) % NBUF, True, True, True)
    for t in range(front + main, nb):
        step(t, t % NBUF, t >= 1, t + 1 < nb, t + 3 < nb)
    wS((nb - 1) % NBUF)


def _slot_scratch(B, ncols):
    sc = []
    for _ in range(NBUF):
        sc += [
            pltpu.VMEM((B,), jnp.int32),
            pltpu.VMEM((B,), jnp.int32),
            pltpu.VMEM((B, ncols), jnp.float32),
            pltpu.SemaphoreType.DMA,
            pltpu.SemaphoreType.DMA,
            pltpu.SemaphoreType.DMA,
            pltpu.SemaphoreType.DMA,
        ]
    return sc


def _sc_agg16(x16, ei):
    B = 200
    ept = E // (NC * NS)
    nb = ept // B
    mesh = plsc.VectorSubcoreMesh(core_axis_name="c", subcore_axis_name="s")

    @functools.partial(
        pl.kernel,
        out_type=jax.ShapeDtypeStruct((NC, N_PAD, 16), jnp.float32),
        mesh=mesh,
        compiler_params=_SC_PARAMS,
        scratch_types=_slot_scratch(B, 16) + [
            pltpu.VMEM((ZR, 16), jnp.float32),
            pltpu.VMEM_SHARED((N_PAD, 16), jnp.float32),
        ],
    )
    def k(x_hbm, ei_hbm, out_hbm, *scr):
        slots = [tuple(scr[i * 7:(i + 1) * 7]) for i in range(NBUF)]
        zbuf, acc = scr[NBUF * 7], scr[NBUF * 7 + 1]
        c = lax.axis_index("c")
        s = lax.axis_index("s")
        _zero_acc(zbuf, acc, s, 16)
        plsc.subcore_barrier()
        base = c * (E // NC) + s * ept
        _edge_pass(x_hbm, ei_hbm, acc, slots, base, nb, B)
        plsc.subcore_barrier()
        pltpu.sync_copy(acc.at[pl.ds(s * RPT, RPT)],
                        out_hbm.at[c].at[pl.ds(s * RPT, RPT)])

    return k(x16, ei)


def _sc_agg128(h4, ei):
    B = 200
    ept = E // NS
    nb = ept // B
    mesh = plsc.VectorSubcoreMesh(core_axis_name="c", subcore_axis_name="s")

    @functools.partial(
        pl.kernel,
        out_type=jax.ShapeDtypeStruct((4, N_PAD, 32), jnp.float32),
        mesh=mesh,
        compiler_params=_SC_PARAMS,
        scratch_types=_slot_scratch(B, 32) + [
            pltpu.VMEM((ZR, 32), jnp.float32),
            pltpu.VMEM_SHARED((N_PAD, 32), jnp.float32),
        ],
    )
    def k(h_hbm, ei_hbm, out_hbm, *scr):
        slots = [tuple(scr[i * 7:(i + 1) * 7]) for i in range(NBUF)]
        zbuf, acc = scr[NBUF * 7], scr[NBUF * 7 + 1]
        c = lax.axis_index("c")
        s = lax.axis_index("s")
        for cc_local in range(2):
            cc = c * 2 + cc_local
            _zero_acc(zbuf, acc, s, 32)
            plsc.subcore_barrier()
            _edge_pass(h_hbm.at[cc], ei_hbm, acc, slots, s * ept, nb, B)
            plsc.subcore_barrier()
            pltpu.sync_copy(acc.at[pl.ds(s * RPT, RPT)],
                            out_hbm.at[cc].at[pl.ds(s * RPT, RPT)])
            plsc.subcore_barrier()

    return k(h4, ei)


BN = 2000


def _tc_layer0(agg0, x16, wl_t, wr_t, b):
    def body(agg_ref, x_ref, wl_ref, wr_ref, b_ref, h4_ref):
        sarr = agg_ref[0] + agg_ref[1]
        cnt = sarr[:, 9:10]
        invc = 1.0 / jnp.maximum(cnt, 1.0)
        m = sarr * invc
        z = jnp.dot(m, wl_ref[...], preferred_element_type=jnp.float32)
        z = z + jnp.dot(x_ref[...], wr_ref[...],
                        preferred_element_type=jnp.float32)
        z = z + b_ref[...]
        h = jnp.maximum(z, 0.0)
        for c4 in range(4):
            h4_ref[c4] = h[:, c4 * 32:(c4 + 1) * 32]

    return pl.pallas_call(
        body,
        grid=(N // BN,),
        in_specs=[
            pl.BlockSpec((2, BN, 16), lambda i: (0, i, 0)),
            pl.BlockSpec((BN, 16), lambda i: (i, 0)),
            pl.BlockSpec((16, H), lambda i: (0, 0)),
            pl.BlockSpec((16, H), lambda i: (0, 0)),
            pl.BlockSpec((1, H), lambda i: (0, 0)),
        ],
        out_specs=pl.BlockSpec((4, BN, 32), lambda i: (0, i, 0)),
        out_shape=jax.ShapeDtypeStruct((4, N, 32), jnp.float32),
    )(agg0, x16, wl_t, wr_t, b)


def _tc_layer(agg4, h4, agg0, wl_t, wr_t, b, relu, out4):
    def body(agg_ref, h_ref, a0_ref, wl_ref, wr_ref, b_ref, out_ref):
        cnt = a0_ref[0, :, 9:10] + a0_ref[1, :, 9:10]
        invc = 1.0 / jnp.maximum(cnt, 1.0)
        m = jnp.concatenate([agg_ref[c] for c in range(4)], axis=1) * invc
        hp = jnp.concatenate([h_ref[c] for c in range(4)], axis=1)
        z = jnp.dot(m, wl_ref[...], preferred_element_type=jnp.float32)
        z = z + jnp.dot(hp, wr_ref[...], preferred_element_type=jnp.float32)
        z = z + b_ref[...]
        if relu:
            z = jnp.maximum(z, 0.0)
        if out4:
            for c4 in range(4):
                out_ref[c4] = z[:, c4 * 32:(c4 + 1) * 32]
        else:
            out_ref[...] = z

    if out4:
        out_spec = pl.BlockSpec((4, BN, 32), lambda i: (0, i, 0))
        out_shape = jax.ShapeDtypeStruct((4, N, 32), jnp.float32)
    else:
        out_spec = pl.BlockSpec((BN, H), lambda i: (i, 0))
        out_shape = jax.ShapeDtypeStruct((N, H), jnp.float32)

    return pl.pallas_call(
        body,
        grid=(N // BN,),
        in_specs=[
            pl.BlockSpec((4, BN, 32), lambda i: (0, i, 0)),
            pl.BlockSpec((4, BN, 32), lambda i: (0, i, 0)),
            pl.BlockSpec((2, BN, 16), lambda i: (0, i, 0)),
            pl.BlockSpec((H, H), lambda i: (0, 0)),
            pl.BlockSpec((H, H), lambda i: (0, 0)),
            pl.BlockSpec((1, H), lambda i: (0, 0)),
        ],
        out_specs=out_spec,
        out_shape=out_shape,
    )(agg4, h4, agg0, wl_t, wr_t, b)


def kernel(x, edge_index, Wl0, Wr0, b0, Wl1, Wr1, b1, Wl2, Wr2, b2):
    x16 = jnp.concatenate(
        [x, jnp.ones((N, 1), jnp.float32), jnp.zeros((N, 6), jnp.float32)],
        axis=1)
    wl0_t = jnp.pad(Wl0, ((0, 0), (0, 7))).T
    wr0_t = jnp.pad(Wr0, ((0, 0), (0, 7))).T
    b0r = b0.reshape(1, H)

    agg0 = _sc_agg16(x16, edge_index)
    h1_4 = _tc_layer0(agg0, x16, wl0_t, wr0_t, b0r)
    agg1 = _sc_agg128(h1_4, edge_index)
    h2_4 = _tc_layer(agg1, h1_4, agg0, Wl1.T, Wr1.T, b1.reshape(1, H),
                     relu=True, out4=True)
    agg2 = _sc_agg128(h2_4, edge_index)
    out = _tc_layer(agg2, h2_4, agg0, Wl2.T, Wr2.T, b2.reshape(1, H),
                    relu=False, out4=False)
    return out


# speedup vs baseline: 10.1872x; 1.0121x over previous
import functools

import jax
import jax.numpy as jnp
from jax import lax
from jax.experimental import pallas as pl
from jax.experimental.pallas import tpu as pltpu
from jax.experimental.pallas import tpu_sc as plsc

N = 50000
E = 800000
H = 128
NC = 2
NS = 16
NBUF = 4
N_PAD = 50176
RPT = N_PAD // NS
ZR = 56

_SC_PARAMS = pltpu.CompilerParams(use_tc_tiling_on_sc=False)


def _fill_zbuf(zbuf, ncols):
    @pl.loop(0, ZR)
    def _(j):
        for k in range(ncols // 16):
            zbuf[j, pl.ds(k * 16, 16)] = jnp.zeros((16,), jnp.float32)


def _zero_acc(zbuf, acc, s, semz):
    @pl.loop(0, RPT // ZR)
    def _(j):
        pltpu.async_copy(zbuf, acc.at[pl.ds(s * RPT + j * ZR, ZR)], semz)

    @pl.loop(0, RPT // ZR)
    def _(j):
        pltpu.make_async_copy(zbuf, acc.at[pl.ds(s * RPT, ZR)], semz).wait()


def _edge_pass(table, ei, acc, slots, base, nb, B):
    def I1(b, k):
        esrc, edst, rows, s1, s2, sg, ss = slots[k]
        pltpu.async_copy(ei.at[0, pl.ds(base + b * B, B)], esrc, s1)

    def wI1(k):
        esrc, edst, rows, s1, s2, sg, ss = slots[k]
        pltpu.make_async_copy(ei.at[0, pl.ds(base, B)], esrc, s1).wait()

    def I2(b, k):
        esrc, edst, rows, s1, s2, sg, ss = slots[k]
        pltpu.async_copy(ei.at[1, pl.ds(base + b * B, B)], edst, s2)

    def wI2(k):
        esrc, edst, rows, s1, s2, sg, ss = slots[k]
        pltpu.make_async_copy(ei.at[1, pl.ds(base, B)], edst, s2).wait()

    def G(k):
        esrc, edst, rows, s1, s2, sg, ss = slots[k]
        pltpu.async_copy(table.at[esrc], rows, sg)

    def wG(k):
        esrc, edst, rows, s1, s2, sg, ss = slots[k]
        pltpu.make_async_copy(table.at[esrc], rows, sg).wait()

    def S(k):
        esrc, edst, rows, s1, s2, sg, ss = slots[k]
        pltpu.async_copy(rows, acc.at[edst], ss, add=True)

    def wS(k):
        esrc, edst, rows, s1, s2, sg, ss = slots[k]
        pltpu.make_async_copy(rows, acc.at[edst], ss).wait()

    def step(tv, k0, g_wsp, g_next, g_i3):
        k1 = (k0 + 1) % NBUF
        k3 = (k0 + 3) % NBUF
        if g_next:
            wI1(k1)
            G(k1)
            I2(tv + 1, k1)
        wG(k0)
        wI2(k0)
        if g_wsp:
            wS(k3)
        S(k0)
        if g_i3:
            I1(tv + 3, k3)

    for b in range(min(3, nb)):
        I1(b, b % NBUF)
    I2(0, 0)
    wI1(0)
    G(0)

    front = min(3, nb)
    main = max(0, nb - 3 - front) // NBUF * NBUF
    for t in range(front):
        step(t, t % NBUF, t >= 1, t + 1 < nb, t + 3 < nb)
    if main > 0:
        @pl.loop(0, main // NBUF)
        def _(g):
            tv = front + g * NBUF
            for j in range(NBUF):
                step(tv + j, (front + j) % NBUF, True, True, True)
    for t in range(front + main, nb):
        step(t, t % NBUF, t >= 1, t + 1 < nb, t + 3 < nb)
    wS((nb - 1) % NBUF)


def _slot_scratch(B, ncols):
    sc = []
    for _ in range(NBUF):
        sc += [
            pltpu.VMEM((B,), jnp.int32),
            pltpu.VMEM((B,), jnp.int32),
            pltpu.VMEM((B, ncols), jnp.float32),
            pltpu.SemaphoreType.DMA,
            pltpu.SemaphoreType.DMA,
            pltpu.SemaphoreType.DMA,
            pltpu.SemaphoreType.DMA,
        ]
    return sc


def _sc_agg16(x16, ei):
    B = 200
    ept = E // (NC * NS)
    nb = ept // B
    mesh = plsc.VectorSubcoreMesh(core_axis_name="c", subcore_axis_name="s")

    @functools.partial(
        pl.kernel,
        out_type=jax.ShapeDtypeStruct((NC, N_PAD, 16), jnp.float32),
        mesh=mesh,
        compiler_params=_SC_PARAMS,
        scratch_types=_slot_scratch(B, 16) + [
            pltpu.VMEM((ZR, 16), jnp.float32),
            pltpu.VMEM_SHARED((N_PAD, 16), jnp.float32),
            pltpu.SemaphoreType.DMA,
        ],
    )
    def k(x_hbm, ei_hbm, out_hbm, *scr):
        slots = [tuple(scr[i * 7:(i + 1) * 7]) for i in range(NBUF)]
        zbuf, acc, semz = scr[NBUF * 7], scr[NBUF * 7 + 1], scr[NBUF * 7 + 2]
        c = lax.axis_index("c")
        s = lax.axis_index("s")
        _fill_zbuf(zbuf, 16)
        _zero_acc(zbuf, acc, s, semz)
        plsc.subcore_barrier()
        base = c * (E // NC) + s * ept
        _edge_pass(x_hbm, ei_hbm, acc, slots, base, nb, B)
        plsc.subcore_barrier()
        pltpu.sync_copy(acc.at[pl.ds(s * RPT, RPT)],
                        out_hbm.at[c].at[pl.ds(s * RPT, RPT)])

    return k(x16, ei)


def _sc_agg128(h4, ei):
    B = 200
    ept = E // NS
    nb = ept // B
    mesh = plsc.VectorSubcoreMesh(core_axis_name="c", subcore_axis_name="s")

    @functools.partial(
        pl.kernel,
        out_type=jax.ShapeDtypeStruct((4, N_PAD, 32), jnp.float32),
        mesh=mesh,
        compiler_params=_SC_PARAMS,
        scratch_types=_slot_scratch(B, 32) + [
            pltpu.VMEM((ZR, 32), jnp.float32),
            pltpu.VMEM_SHARED((N_PAD, 32), jnp.float32),
            pltpu.SemaphoreType.DMA,
        ],
    )
    def k(h_hbm, ei_hbm, out_hbm, *scr):
        slots = [tuple(scr[i * 7:(i + 1) * 7]) for i in range(NBUF)]
        zbuf, acc, semz = scr[NBUF * 7], scr[NBUF * 7 + 1], scr[NBUF * 7 + 2]
        c = lax.axis_index("c")
        s = lax.axis_index("s")
        _fill_zbuf(zbuf, 32)
        _zero_acc(zbuf, acc, s, semz)
        plsc.subcore_barrier()
        for cc_local in range(2):
            cc = c * 2 + cc_local
            _edge_pass(h_hbm.at[cc], ei_hbm, acc, slots, s * ept, nb, B)
            plsc.subcore_barrier()
            pltpu.sync_copy(acc.at[pl.ds(s * RPT, RPT)],
                            out_hbm.at[cc].at[pl.ds(s * RPT, RPT)])
            if cc_local == 0:
                _zero_acc(zbuf, acc, s, semz)
            plsc.subcore_barrier()

    return k(h4, ei)


BN = 2000


def _tc_layer0(agg0, x16, wl_t, wr_t, b):
    def body(agg_ref, x_ref, wl_ref, wr_ref, b_ref, h4_ref):
        sarr = agg_ref[0] + agg_ref[1]
        cnt = sarr[:, 9:10]
        invc = 1.0 / jnp.maximum(cnt, 1.0)
        m = sarr * invc
        z = jnp.dot(m, wl_ref[...], preferred_element_type=jnp.float32)
        z = z + jnp.dot(x_ref[...], wr_ref[...],
                        preferred_element_type=jnp.float32)
        z = z + b_ref[...]
        h = jnp.maximum(z, 0.0)
        for c4 in range(4):
            h4_ref[c4] = h[:, c4 * 32:(c4 + 1) * 32]

    return pl.pallas_call(
        body,
        grid=(N // BN,),
        in_specs=[
            pl.BlockSpec((2, BN, 16), lambda i: (0, i, 0)),
            pl.BlockSpec((BN, 16), lambda i: (i, 0)),
            pl.BlockSpec((16, H), lambda i: (0, 0)),
            pl.BlockSpec((16, H), lambda i: (0, 0)),
            pl.BlockSpec((1, H), lambda i: (0, 0)),
        ],
        out_specs=pl.BlockSpec((4, BN, 32), lambda i: (0, i, 0)),
        out_shape=jax.ShapeDtypeStruct((4, N, 32), jnp.float32),
    )(agg0, x16, wl_t, wr_t, b)


def _tc_layer(agg4, h4, agg0, wl_t, wr_t, b, relu, out4):
    def body(agg_ref, h_ref, a0_ref, wl_ref, wr_ref, b_ref, out_ref):
        cnt = a0_ref[0, :, 9:10] + a0_ref[1, :, 9:10]
        invc = 1.0 / jnp.maximum(cnt, 1.0)
        m = jnp.concatenate([agg_ref[c] for c in range(4)], axis=1) * invc
        hp = jnp.concatenate([h_ref[c] for c in range(4)], axis=1)
        z = jnp.dot(m, wl_ref[...], preferred_element_type=jnp.float32)
        z = z + jnp.dot(hp, wr_ref[...], preferred_element_type=jnp.float32)
        z = z + b_ref[...]
        if relu:
            z = jnp.maximum(z, 0.0)
        if out4:
            for c4 in range(4):
                out_ref[c4] = z[:, c4 * 32:(c4 + 1) * 32]
        else:
            out_ref[...] = z

    if out4:
        out_spec = pl.BlockSpec((4, BN, 32), lambda i: (0, i, 0))
        out_shape = jax.ShapeDtypeStruct((4, N, 32), jnp.float32)
    else:
        out_spec = pl.BlockSpec((BN, H), lambda i: (i, 0))
        out_shape = jax.ShapeDtypeStruct((N, H), jnp.float32)

    return pl.pallas_call(
        body,
        grid=(N // BN,),
        in_specs=[
            pl.BlockSpec((4, BN, 32), lambda i: (0, i, 0)),
            pl.BlockSpec((4, BN, 32), lambda i: (0, i, 0)),
            pl.BlockSpec((2, BN, 16), lambda i: (0, i, 0)),
            pl.BlockSpec((H, H), lambda i: (0, 0)),
            pl.BlockSpec((H, H), lambda i: (0, 0)),
            pl.BlockSpec((1, H), lambda i: (0, 0)),
        ],
        out_specs=out_spec,
        out_shape=out_shape,
    )(agg4, h4, agg0, wl_t, wr_t, b)


def kernel(x, edge_index, Wl0, Wr0, b0, Wl1, Wr1, b1, Wl2, Wr2, b2):
    x16 = jnp.concatenate(
        [x, jnp.ones((N, 1), jnp.float32), jnp.zeros((N, 6), jnp.float32)],
        axis=1)
    wl0_t = jnp.pad(Wl0, ((0, 0), (0, 7))).T
    wr0_t = jnp.pad(Wr0, ((0, 0), (0, 7))).T
    b0r = b0.reshape(1, H)

    agg0 = _sc_agg16(x16, edge_index)
    h1_4 = _tc_layer0(agg0, x16, wl0_t, wr0_t, b0r)
    agg1 = _sc_agg128(h1_4, edge_index)
    h2_4 = _tc_layer(agg1, h1_4, agg0, Wl1.T, Wr1.T, b1.reshape(1, H),
                     relu=True, out4=True)
    agg2 = _sc_agg128(h2_4, edge_index)
    out = _tc_layer(agg2, h2_4, agg0, Wl2.T, Wr2.T, b2.reshape(1, H),
                    relu=False, out4=False)
    return out


# speedup vs baseline: 11.5437x; 1.1332x over previous
import functools

import jax
import jax.numpy as jnp
from jax import lax
from jax.experimental import pallas as pl
from jax.experimental.pallas import tpu as pltpu
from jax.experimental.pallas import tpu_sc as plsc

N = 50000
E = 800000
H = 128
NC = 2
NS = 16
NBUF = 4
N_PAD = 50176
RPT = N_PAD // NS
ZR = 56

_SC_PARAMS = pltpu.CompilerParams(use_tc_tiling_on_sc=False)


def _fill_zbuf(zbuf, ncols):
    @pl.loop(0, ZR)
    def _(j):
        for k in range(ncols // 16):
            zbuf[j, pl.ds(k * 16, 16)] = jnp.zeros((16,), jnp.float32)


def _zero_acc(zbuf, acc, s, semz):
    @pl.loop(0, RPT // ZR)
    def _(j):
        pltpu.async_copy(zbuf, acc.at[pl.ds(s * RPT + j * ZR, ZR)], semz)

    @pl.loop(0, RPT // ZR)
    def _(j):
        pltpu.make_async_copy(zbuf, acc.at[pl.ds(s * RPT, ZR)], semz).wait()


def _edge_pass(table, ei, acc, islots, rslots, base, nb, B):
    def I(b, j):
        eidx, si = islots[j]
        pltpu.async_copy(ei.at[:, pl.ds(base + b * B, B)], eidx, si)

    def wI(j):
        eidx, si = islots[j]
        pltpu.make_async_copy(ei.at[:, pl.ds(base, B)], eidx, si).wait()

    def G(j, k):
        eidx, si = islots[j]
        rows, sg, ss = rslots[k]
        pltpu.async_copy(table.at[eidx.at[0]], rows, sg)

    def wG(j, k):
        eidx, si = islots[j]
        rows, sg, ss = rslots[k]
        pltpu.make_async_copy(table.at[eidx.at[0]], rows, sg).wait()

    def S(j, k):
        eidx, si = islots[j]
        rows, sg, ss = rslots[k]
        pltpu.async_copy(rows, acc.at[eidx.at[1]], ss, add=True)

    def wS(j, k):
        eidx, si = islots[j]
        rows, sg, ss = rslots[k]
        pltpu.make_async_copy(rows, acc.at[eidx.at[1]], ss).wait()

    def step(tv, t8, g_ws, g_t3, g_i7):
        j0, k0 = t8, t8 % 4
        jm1, km1 = (t8 + 7) % 8, (t8 + 3) % 4
        j3, k3 = (t8 + 3) % 8, (t8 + 3) % 4
        wG(j0, k0)
        if g_ws:
            wS(jm1, km1)
        S(j0, k0)
        if g_i7:
            I(tv + 7, jm1)
        if g_t3:
            wI(j3)
            G(j3, k3)

    for b in range(min(7, nb)):
        I(b, b % 8)
    for b in range(min(3, nb)):
        wI(b % 8)
        G(b % 8, b % 4)

    front = 1
    main = max(0, nb - 7 - front) // 8 * 8
    for t in range(front):
        step(t, t % 8, t >= 1, t + 3 < nb, t + 7 < nb)
    if main > 0:
        @pl.loop(0, main // 8)
        def _(g):
            tv = front + g * 8
            for j in range(8):
                step(tv + j, (front + j) % 8, True, True, True)
    for t in range(front + main, nb):
        step(t, t % 8, t >= 1, t + 3 < nb, t + 7 < nb)
    wS((nb - 1) % 8, (nb - 1) % 4)


def _slot_scratch(B, ncols):
    sc = []
    for _ in range(8):
        sc += [pltpu.VMEM((2, B), jnp.int32), pltpu.SemaphoreType.DMA]
    for _ in range(4):
        sc += [pltpu.VMEM((B, ncols), jnp.float32),
               pltpu.SemaphoreType.DMA, pltpu.SemaphoreType.DMA]
    return sc


def _unpack_slots(scr):
    islots = [tuple(scr[2 * i:2 * i + 2]) for i in range(8)]
    rslots = [tuple(scr[16 + 3 * i:16 + 3 * i + 3]) for i in range(4)]
    return islots, rslots, scr[28:]


def _sc_agg16(x16, ei):
    B = 200
    ept = E // (NC * NS)
    nb = ept // B
    mesh = plsc.VectorSubcoreMesh(core_axis_name="c", subcore_axis_name="s")

    @functools.partial(
        pl.kernel,
        out_type=jax.ShapeDtypeStruct((NC, N_PAD, 16), jnp.float32),
        mesh=mesh,
        compiler_params=_SC_PARAMS,
        scratch_types=_slot_scratch(B, 16) + [
            pltpu.VMEM((ZR, 16), jnp.float32),
            pltpu.VMEM_SHARED((N_PAD, 16), jnp.float32),
            pltpu.SemaphoreType.DMA,
        ],
    )
    def k(x_hbm, ei_hbm, out_hbm, *scr):
        islots, rslots, rest = _unpack_slots(scr)
        zbuf, acc, semz = rest
        c = lax.axis_index("c")
        s = lax.axis_index("s")
        _fill_zbuf(zbuf, 16)
        _zero_acc(zbuf, acc, s, semz)
        plsc.subcore_barrier()
        base = c * (E // NC) + s * ept
        _edge_pass(x_hbm, ei_hbm, acc, islots, rslots, base, nb, B)
        plsc.subcore_barrier()
        pltpu.sync_copy(acc.at[pl.ds(s * RPT, RPT)],
                        out_hbm.at[c].at[pl.ds(s * RPT, RPT)])

    return k(x16, ei)


def _sc_agg128(h4, ei):
    B = 200
    ept = E // NS
    nb = ept // B
    mesh = plsc.VectorSubcoreMesh(core_axis_name="c", subcore_axis_name="s")

    @functools.partial(
        pl.kernel,
        out_type=jax.ShapeDtypeStruct((4, N_PAD, 32), jnp.float32),
        mesh=mesh,
        compiler_params=_SC_PARAMS,
        scratch_types=_slot_scratch(B, 32) + [
            pltpu.VMEM((ZR, 32), jnp.float32),
            pltpu.VMEM_SHARED((N_PAD, 32), jnp.float32),
            pltpu.SemaphoreType.DMA,
        ],
    )
    def k(h_hbm, ei_hbm, out_hbm, *scr):
        islots, rslots, rest = _unpack_slots(scr)
        zbuf, acc, semz = rest
        c = lax.axis_index("c")
        s = lax.axis_index("s")
        _fill_zbuf(zbuf, 32)
        _zero_acc(zbuf, acc, s, semz)
        plsc.subcore_barrier()
        for cc_local in range(2):
            cc = c * 2 + cc_local
            _edge_pass(h_hbm.at[cc], ei_hbm, acc, islots, rslots,
                       s * ept, nb, B)
            plsc.subcore_barrier()
            pltpu.sync_copy(acc.at[pl.ds(s * RPT, RPT)],
                            out_hbm.at[cc].at[pl.ds(s * RPT, RPT)])
            if cc_local == 0:
                _zero_acc(zbuf, acc, s, semz)
            plsc.subcore_barrier()

    return k(h4, ei)


BN = 2000


def _tc_layer0(agg0, x16, wl_t, wr_t, b):
    def body(agg_ref, x_ref, wl_ref, wr_ref, b_ref, h4_ref):
        sarr = agg_ref[0] + agg_ref[1]
        cnt = sarr[:, 9:10]
        invc = 1.0 / jnp.maximum(cnt, 1.0)
        m = sarr * invc
        z = jnp.dot(m, wl_ref[...], preferred_element_type=jnp.float32)
        z = z + jnp.dot(x_ref[...], wr_ref[...],
                        preferred_element_type=jnp.float32)
        z = z + b_ref[...]
        h = jnp.maximum(z, 0.0)
        for c4 in range(4):
            h4_ref[c4] = h[:, c4 * 32:(c4 + 1) * 32]

    return pl.pallas_call(
        body,
        grid=(N // BN,),
        in_specs=[
            pl.BlockSpec((2, BN, 16), lambda i: (0, i, 0)),
            pl.BlockSpec((BN, 16), lambda i: (i, 0)),
            pl.BlockSpec((16, H), lambda i: (0, 0)),
            pl.BlockSpec((16, H), lambda i: (0, 0)),
            pl.BlockSpec((1, H), lambda i: (0, 0)),
        ],
        out_specs=pl.BlockSpec((4, BN, 32), lambda i: (0, i, 0)),
        out_shape=jax.ShapeDtypeStruct((4, N, 32), jnp.float32),
    )(agg0, x16, wl_t, wr_t, b)


def _tc_layer(agg4, h4, agg0, wl_t, wr_t, b, relu, out4):
    def body(agg_ref, h_ref, a0_ref, wl_ref, wr_ref, b_ref, out_ref):
        cnt = a0_ref[0, :, 9:10] + a0_ref[1, :, 9:10]
        invc = 1.0 / jnp.maximum(cnt, 1.0)
        m = jnp.concatenate([agg_ref[c] for c in range(4)], axis=1) * invc
        hp = jnp.concatenate([h_ref[c] for c in range(4)], axis=1)
        z = jnp.dot(m, wl_ref[...], preferred_element_type=jnp.float32)
        z = z + jnp.dot(hp, wr_ref[...], preferred_element_type=jnp.float32)
        z = z + b_ref[...]
        if relu:
            z = jnp.maximum(z, 0.0)
        if out4:
            for c4 in range(4):
                out_ref[c4] = z[:, c4 * 32:(c4 + 1) * 32]
        else:
            out_ref[...] = z

    if out4:
        out_spec = pl.BlockSpec((4, BN, 32), lambda i: (0, i, 0))
        out_shape = jax.ShapeDtypeStruct((4, N, 32), jnp.float32)
    else:
        out_spec = pl.BlockSpec((BN, H), lambda i: (i, 0))
        out_shape = jax.ShapeDtypeStruct((N, H), jnp.float32)

    return pl.pallas_call(
        body,
        grid=(N // BN,),
        in_specs=[
            pl.BlockSpec((4, BN, 32), lambda i: (0, i, 0)),
            pl.BlockSpec((4, BN, 32), lambda i: (0, i, 0)),
            pl.BlockSpec((2, BN, 16), lambda i: (0, i, 0)),
            pl.BlockSpec((H, H), lambda i: (0, 0)),
            pl.BlockSpec((H, H), lambda i: (0, 0)),
            pl.BlockSpec((1, H), lambda i: (0, 0)),
        ],
        out_specs=out_spec,
        out_shape=out_shape,
    )(agg4, h4, agg0, wl_t, wr_t, b)


def kernel(x, edge_index, Wl0, Wr0, b0, Wl1, Wr1, b1, Wl2, Wr2, b2):
    x16 = jnp.concatenate(
        [x, jnp.ones((N, 1), jnp.float32), jnp.zeros((N, 6), jnp.float32)],
        axis=1)
    wl0_t = jnp.pad(Wl0, ((0, 0), (0, 7))).T
    wr0_t = jnp.pad(Wr0, ((0, 0), (0, 7))).T
    b0r = b0.reshape(1, H)

    agg0 = _sc_agg16(x16, edge_index)
    h1_4 = _tc_layer0(agg0, x16, wl0_t, wr0_t, b0r)
    agg1 = _sc_agg128(h1_4, edge_index)
    h2_4 = _tc_layer(agg1, h1_4, agg0, Wl1.T, Wr1.T, b1.reshape(1, H),
                     relu=True, out4=True)
    agg2 = _sc_agg128(h2_4, edge_index)
    out = _tc_layer(agg2, h2_4, agg0, Wl2.T, Wr2.T, b2.reshape(1, H),
                    relu=False, out4=False)
    return out
